# Initial kernel scaffold; baseline (speedup 1.0000x reference)
import jax
import jax.numpy as jnp
from jax.experimental import pallas as pl


def kernel(x, edge_index, edge_type, cell_dimensions, W1, b1, W2, b2, Wf1, bf1, Wf2, bf2):
    raise NotImplementedError("write your pallas kernel here")


# speedup vs baseline: 4.7901x; 4.7901x over previous
import functools

import jax
import jax.numpy as jnp
from jax import lax
from jax.experimental import pallas as pl
from jax.experimental.pallas import tpu as pltpu
from jax.experimental.pallas import tpu_sc as plsc

N = 10000
E = 160000
D = 256
R = 3

SLAB = 10240
ACC_ROWS = 3 * SLAB
CBLK = 32
NCB = D // CBLK
NTILES = 16
K = 128
CHUNKS = 79
EPT = CHUNKS * K
E_PAD = NTILES * EPT
ZROWS = 1920
DUMMY_ROW = N


def _sc_scatter_kernel(x4_hbm, g_hbm, s_hbm, z_hbm, out_hbm,
                       gidx_v, sidx_v, rows_v, zbuf_v, acc_sh, sem):
    cid = lax.axis_index("c")
    sid = lax.axis_index("s")
    row0 = sid * ZROWS

    pltpu.sync_copy(s_hbm.at[sid], sidx_v)
    pltpu.sync_copy(z_hbm, zbuf_v)

    for p in range(NCB // 2):
        for c in range(2):
            @pl.when(cid == c)
            def _load_gidx(p=p, c=c):
                pltpu.sync_copy(g_hbm.at[2 * p + c, sid], gidx_v)

        for k in range(ZROWS // K):
            pltpu.sync_copy(zbuf_v, acc_sh.at[pl.ds(row0 + K * k, K)])
        plsc.subcore_barrier()

        @pl.loop(0, CHUNKS)
        def _chunk(j):
            pltpu.async_copy(x4_hbm.at[gidx_v.at[j]], rows_v, sem).wait()
            pltpu.sync_copy(rows_v, acc_sh.at[sidx_v.at[j]], add=True)

        plsc.subcore_barrier()

        for c in range(2):
            @pl.when(cid == c)
            def _drain(p=p, c=c):
                cb = 2 * p + c
                pltpu.sync_copy(
                    acc_sh.at[pl.ds(row0, ZROWS)],
                    out_hbm.at[pl.ds(row0, ZROWS), pl.ds(cb * CBLK, CBLK)])

        if p != NCB // 2 - 1:
            plsc.subcore_barrier()


def _sc_scatter(x, gidx, sidx):
    x4 = x.reshape(N * NCB, CBLK)
    zeros = jnp.zeros((K, CBLK), jnp.float32)
    mesh = plsc.VectorSubcoreMesh(core_axis_name="c", subcore_axis_name="s")
    f = pl.kernel(
        _sc_scatter_kernel,
        out_type=jax.ShapeDtypeStruct((ACC_ROWS, D), jnp.float32),
        mesh=mesh,
        scratch_types=[
            pltpu.VMEM((CHUNKS, K), jnp.int32),
            pltpu.VMEM((CHUNKS, K), jnp.int32),
            pltpu.VMEM((K, CBLK), jnp.float32),
            pltpu.VMEM((K, CBLK), jnp.float32),
            pltpu.VMEM_SHARED((ACC_ROWS, CBLK), jnp.float32),
            pltpu.SemaphoreType.DMA,
        ],
        compiler_params=pltpu.CompilerParams(use_tc_tiling_on_sc=False),
    )
    return f(x4, gidx, sidx, zeros)


def _mlp_kernel(acc_ref, x_ref, w1_ref, b1_ref, w2_ref, b2_ref,
                wf1_ref, bf1_ref, wf2_ref, bf2_ref, out_ref):
    xr = x_ref[...]
    f = jnp.zeros_like(xr)
    for r in range(R):
        m = acc_ref[r] + xr
        h = jnp.maximum(jnp.dot(m, w1_ref[r],
                                preferred_element_type=jnp.float32) + b1_ref[r], 0.0)
        h = jnp.maximum(jnp.dot(h, w2_ref[r],
                                preferred_element_type=jnp.float32) + b2_ref[r], 0.0)
        f = f + jnp.dot(h, wf1_ref[r], preferred_element_type=jnp.float32)
    g = jnp.maximum(f + bf1_ref[...], 0.0)
    out_ref[...] = jnp.maximum(
        jnp.dot(g, wf2_ref[...], preferred_element_type=jnp.float32)
        + bf2_ref[...], 0.0)


def _mlp(acc, x, W1, b1, W2, b2, Wf1, bf1, Wf2, bf2):
    BN = 1000
    grid = (N // BN,)
    acc3 = acc.reshape(R, SLAB, D)
    return pl.pallas_call(
        _mlp_kernel,
        grid=grid,
        in_specs=[
            pl.BlockSpec((R, BN, D), lambda i: (0, i, 0)),
            pl.BlockSpec((BN, D), lambda i: (i, 0)),
            pl.BlockSpec((R, D, D), lambda i: (0, 0, 0)),
            pl.BlockSpec((R, 1, D), lambda i: (0, 0, 0)),
            pl.BlockSpec((R, D, D), lambda i: (0, 0, 0)),
            pl.BlockSpec((R, 1, D), lambda i: (0, 0, 0)),
            pl.BlockSpec((R, D, D), lambda i: (0, 0, 0)),
            pl.BlockSpec((1, D), lambda i: (0, 0)),
            pl.BlockSpec((D, D), lambda i: (0, 0)),
            pl.BlockSpec((1, D), lambda i: (0, 0)),
        ],
        out_specs=pl.BlockSpec((BN, D), lambda i: (i, 0)),
        out_shape=jax.ShapeDtypeStruct((N, D), jnp.float32),
    )(acc3, x, W1, b1.reshape(R, 1, D), W2, b2.reshape(R, 1, D),
      Wf1.reshape(R, D, D), bf1.reshape(1, D), Wf2, bf2.reshape(1, D))


def kernel(x, edge_index, edge_type, cell_dimensions,
           W1, b1, W2, b2, Wf1, bf1, Wf2, bf2):
    del cell_dimensions
    src = edge_index[0]
    dst = edge_index[1]
    pad = E_PAD - E
    srcp = jnp.concatenate([src, jnp.zeros((pad,), jnp.int32)])
    comb = edge_type * SLAB + dst
    combp = jnp.concatenate([comb, jnp.full((pad,), DUMMY_ROW, jnp.int32)])
    gidx = (srcp * NCB)[None, :] + jnp.arange(NCB, dtype=jnp.int32)[:, None]
    gidx = gidx.reshape(NCB, NTILES, CHUNKS, K)
    sidx = combp.reshape(NTILES, CHUNKS, K)

    acc = _sc_scatter(x, gidx, sidx)
    return _mlp(acc, x, W1, b1, W2, b2, Wf1, bf1, Wf2, bf2)


# speedup vs baseline: 4.9571x; 1.0349x over previous
import functools

import jax
import jax.numpy as jnp
from jax import lax
from jax.experimental import pallas as pl
from jax.experimental.pallas import tpu as pltpu
from jax.experimental.pallas import tpu_sc as plsc

N = 10000
E = 160000
D = 256
R = 3

SLAB = 10240
ACC_ROWS = 3 * SLAB
CBLK = 32
NCB = D // CBLK
NTILES = 16
K = 128
CHUNKS = 80
EPT = CHUNKS * K
E_PAD = NTILES * EPT
NBUF = 4
ZROWS = 1920
DUMMY_ROW = N


def _sc_scatter_kernel(x4_hbm, g_hbm, s_hbm, z_hbm, out_hbm,
                       gidx_v, sidx_v, bufs_v, zbuf_v, acc_sh, gsem, ssem, zsem):
    cid = lax.axis_index("c")
    sid = lax.axis_index("s")
    row0 = sid * ZROWS

    pltpu.sync_copy(s_hbm.at[sid], sidx_v)
    pltpu.sync_copy(z_hbm, zbuf_v)

    def gather_issue(j, b):
        return pltpu.async_copy(x4_hbm.at[gidx_v.at[j]], bufs_v.at[b],
                                gsem.at[b])

    def gather_wait(j, b):
        pltpu.make_async_copy(x4_hbm.at[gidx_v.at[j]], bufs_v.at[b],
                              gsem.at[b]).wait()

    def scatter_issue(j, b):
        return pltpu.async_copy(bufs_v.at[b], acc_sh.at[sidx_v.at[j]],
                                ssem.at[b], add=True)

    def scatter_wait(j, b):
        pltpu.make_async_copy(bufs_v.at[b], acc_sh.at[sidx_v.at[j]],
                              ssem.at[b]).wait()

    for p in range(NCB // 2):
        for c in range(2):
            @pl.when(cid == c)
            def _load_gidx(p=p, c=c):
                pltpu.sync_copy(g_hbm.at[2 * p + c, sid], gidx_v)

        gather_issue(0, 0)
        gather_issue(1, 1)
        for k in range(ZROWS // K):
            pltpu.async_copy(zbuf_v, acc_sh.at[pl.ds(row0 + K * k, K)], zsem)
        for k in range(ZROWS // K):
            pltpu.make_async_copy(zbuf_v, acc_sh.at[pl.ds(K, K)], zsem).wait()
        plsc.subcore_barrier()

        @pl.loop(0, CHUNKS // NBUF)
        def _group(i):
            for b in range(NBUF):
                j = NBUF * i + b
                b2 = (b + 2) % NBUF
                gather_wait(j, b)
                scatter_issue(j, b)

                @pl.when(j >= 2)
                def _retire():
                    scatter_wait(j - 2, b2)

                @pl.when(j + 2 < CHUNKS)
                def _prefetch():
                    gather_issue(j + 2, b2)

        scatter_wait(CHUNKS - 2, (CHUNKS - 2) % NBUF)
        scatter_wait(CHUNKS - 1, (CHUNKS - 1) % NBUF)
        plsc.subcore_barrier()

        for c in range(2):
            @pl.when(cid == c)
            def _drain(p=p, c=c):
                cb = 2 * p + c
                pltpu.sync_copy(
                    acc_sh.at[pl.ds(row0, ZROWS)],
                    out_hbm.at[pl.ds(row0, ZROWS), pl.ds(cb * CBLK, CBLK)])

        if p != NCB // 2 - 1:
            plsc.subcore_barrier()


def _sc_scatter(x, gidx, sidx):
    x4 = x.reshape(N * NCB, CBLK)
    zeros = jnp.zeros((K, CBLK), jnp.float32)
    mesh = plsc.VectorSubcoreMesh(core_axis_name="c", subcore_axis_name="s")
    f = pl.kernel(
        _sc_scatter_kernel,
        out_type=jax.ShapeDtypeStruct((ACC_ROWS, D), jnp.float32),
        mesh=mesh,
        scratch_types=[
            pltpu.VMEM((CHUNKS, K), jnp.int32),
            pltpu.VMEM((CHUNKS, K), jnp.int32),
            pltpu.VMEM((NBUF, K, CBLK), jnp.float32),
            pltpu.VMEM((K, CBLK), jnp.float32),
            pltpu.VMEM_SHARED((ACC_ROWS, CBLK), jnp.float32),
            pltpu.SemaphoreType.DMA((NBUF,)),
            pltpu.SemaphoreType.DMA((NBUF,)),
            pltpu.SemaphoreType.DMA,
        ],
        compiler_params=pltpu.CompilerParams(use_tc_tiling_on_sc=False),
    )
    return f(x4, gidx, sidx, zeros)


def _mlp_kernel(acc_ref, x_ref, w1_ref, b1_ref, w2_ref, b2_ref,
                wf1_ref, bf1_ref, wf2_ref, bf2_ref, out_ref):
    xr = x_ref[...]
    f = jnp.zeros_like(xr)
    for r in range(R):
        m = acc_ref[r] + xr
        h = jnp.maximum(jnp.dot(m, w1_ref[r],
                                preferred_element_type=jnp.float32) + b1_ref[r], 0.0)
        h = jnp.maximum(jnp.dot(h, w2_ref[r],
                                preferred_element_type=jnp.float32) + b2_ref[r], 0.0)
        f = f + jnp.dot(h, wf1_ref[r], preferred_element_type=jnp.float32)
    g = jnp.maximum(f + bf1_ref[...], 0.0)
    out_ref[...] = jnp.maximum(
        jnp.dot(g, wf2_ref[...], preferred_element_type=jnp.float32)
        + bf2_ref[...], 0.0)


def _mlp(acc, x, W1, b1, W2, b2, Wf1, bf1, Wf2, bf2):
    BN = 1000
    grid = (N // BN,)
    acc3 = acc.reshape(R, SLAB, D)
    return pl.pallas_call(
        _mlp_kernel,
        grid=grid,
        in_specs=[
            pl.BlockSpec((R, BN, D), lambda i: (0, i, 0)),
            pl.BlockSpec((BN, D), lambda i: (i, 0)),
            pl.BlockSpec((R, D, D), lambda i: (0, 0, 0)),
            pl.BlockSpec((R, 1, D), lambda i: (0, 0, 0)),
            pl.BlockSpec((R, D, D), lambda i: (0, 0, 0)),
            pl.BlockSpec((R, 1, D), lambda i: (0, 0, 0)),
            pl.BlockSpec((R, D, D), lambda i: (0, 0, 0)),
            pl.BlockSpec((1, D), lambda i: (0, 0)),
            pl.BlockSpec((D, D), lambda i: (0, 0)),
            pl.BlockSpec((1, D), lambda i: (0, 0)),
        ],
        out_specs=pl.BlockSpec((BN, D), lambda i: (i, 0)),
        out_shape=jax.ShapeDtypeStruct((N, D), jnp.float32),
    )(acc3, x, W1, b1.reshape(R, 1, D), W2, b2.reshape(R, 1, D),
      Wf1.reshape(R, D, D), bf1.reshape(1, D), Wf2, bf2.reshape(1, D))


def kernel(x, edge_index, edge_type, cell_dimensions,
           W1, b1, W2, b2, Wf1, bf1, Wf2, bf2):
    del cell_dimensions
    src = edge_index[0]
    dst = edge_index[1]
    pad = E_PAD - E
    srcp = jnp.concatenate([src, jnp.zeros((pad,), jnp.int32)])
    comb = edge_type * SLAB + dst
    combp = jnp.concatenate([comb, jnp.full((pad,), DUMMY_ROW, jnp.int32)])
    gidx = (srcp * NCB)[None, :] + jnp.arange(NCB, dtype=jnp.int32)[:, None]
    gidx = gidx.reshape(NCB, NTILES, CHUNKS, K)
    sidx = combp.reshape(NTILES, CHUNKS, K)

    acc = _sc_scatter(x, gidx, sidx)
    return _mlp(acc, x, W1, b1, W2, b2, Wf1, bf1, Wf2, bf2)


# speedup vs baseline: 7.9076x; 1.5952x over previous
import functools

import jax
import jax.numpy as jnp
from jax import lax
from jax.experimental import pallas as pl
from jax.experimental.pallas import tpu as pltpu
from jax.experimental.pallas import tpu_sc as plsc

N = 10000
E = 160000
D = 256
R = 3

SLAB = 10240
ACC_ROWS = 3 * SLAB
CBLK = 64
NCB = D // CBLK
ACC_DT = jnp.bfloat16
NTILES = 16
K = 128
CHUNKS = 80
EPT = CHUNKS * K
E_PAD = NTILES * EPT
NBUF = 4
ZROWS = 1920
DUMMY_ROW = N


def _sc_scatter_kernel(x4_hbm, g_hbm, s_hbm, z_hbm, out_hbm,
                       gidx_v, sidx_v, bufs_v, zbuf_v, acc_sh, gsem, ssem, zsem):
    cid = lax.axis_index("c")
    sid = lax.axis_index("s")
    row0 = sid * ZROWS

    pltpu.sync_copy(s_hbm.at[sid], sidx_v)
    pltpu.sync_copy(z_hbm, zbuf_v)

    def gather_issue(j, b):
        return pltpu.async_copy(x4_hbm.at[gidx_v.at[j]], bufs_v.at[b],
                                gsem.at[b])

    def gather_wait(j, b):
        pltpu.make_async_copy(x4_hbm.at[gidx_v.at[j]], bufs_v.at[b],
                              gsem.at[b]).wait()

    def scatter_issue(j, b):
        return pltpu.async_copy(bufs_v.at[b], acc_sh.at[sidx_v.at[j]],
                                ssem.at[b], add=True)

    def scatter_wait(j, b):
        pltpu.make_async_copy(bufs_v.at[b], acc_sh.at[sidx_v.at[j]],
                              ssem.at[b]).wait()

    for p in range(NCB // 2):
        for c in range(2):
            @pl.when(cid == c)
            def _load_gidx(p=p, c=c):
                pltpu.sync_copy(g_hbm.at[2 * p + c, sid], gidx_v)

        gather_issue(0, 0)
        gather_issue(1, 1)
        for k in range(ZROWS // K):
            pltpu.async_copy(zbuf_v, acc_sh.at[pl.ds(row0 + K * k, K)], zsem)
        for k in range(ZROWS // K):
            pltpu.make_async_copy(zbuf_v, acc_sh.at[pl.ds(K, K)], zsem).wait()
        plsc.subcore_barrier()

        @pl.loop(0, CHUNKS // NBUF)
        def _group(i):
            for b in range(NBUF):
                j = NBUF * i + b
                b2 = (b + 2) % NBUF
                gather_wait(j, b)
                scatter_issue(j, b)

                @pl.when(j >= 2)
                def _retire():
                    scatter_wait(j - 2, b2)

                @pl.when(j + 2 < CHUNKS)
                def _prefetch():
                    gather_issue(j + 2, b2)

        scatter_wait(CHUNKS - 2, (CHUNKS - 2) % NBUF)
        scatter_wait(CHUNKS - 1, (CHUNKS - 1) % NBUF)
        plsc.subcore_barrier()

        for c in range(2):
            @pl.when(cid == c)
            def _drain(p=p, c=c):
                cb = 2 * p + c
                pltpu.sync_copy(
                    acc_sh.at[pl.ds(row0, ZROWS)],
                    out_hbm.at[pl.ds(row0, ZROWS), pl.ds(cb * CBLK, CBLK)])

        if p != NCB // 2 - 1:
            plsc.subcore_barrier()


def _sc_scatter(x, gidx, sidx):
    x4 = x.astype(ACC_DT).reshape(N * NCB, CBLK)
    zeros = jnp.zeros((K, CBLK), ACC_DT)
    mesh = plsc.VectorSubcoreMesh(core_axis_name="c", subcore_axis_name="s")
    f = pl.kernel(
        _sc_scatter_kernel,
        out_type=jax.ShapeDtypeStruct((ACC_ROWS, D), ACC_DT),
        mesh=mesh,
        scratch_types=[
            pltpu.VMEM((CHUNKS, K), jnp.int32),
            pltpu.VMEM((CHUNKS, K), jnp.int32),
            pltpu.VMEM((NBUF, K, CBLK), ACC_DT),
            pltpu.VMEM((K, CBLK), ACC_DT),
            pltpu.VMEM_SHARED((ACC_ROWS, CBLK), ACC_DT),
            pltpu.SemaphoreType.DMA((NBUF,)),
            pltpu.SemaphoreType.DMA((NBUF,)),
            pltpu.SemaphoreType.DMA,
        ],
        compiler_params=pltpu.CompilerParams(use_tc_tiling_on_sc=False),
    )
    return f(x4, gidx, sidx, zeros)


def _mlp_kernel(acc_ref, x_ref, w1_ref, b1_ref, w2_ref, b2_ref,
                wf1_ref, bf1_ref, wf2_ref, bf2_ref, out_ref):
    xr = x_ref[...]
    f = jnp.zeros_like(xr)
    for r in range(R):
        m = acc_ref[r].astype(jnp.float32) + xr
        h = jnp.maximum(jnp.dot(m, w1_ref[r],
                                preferred_element_type=jnp.float32) + b1_ref[r], 0.0)
        h = jnp.maximum(jnp.dot(h, w2_ref[r],
                                preferred_element_type=jnp.float32) + b2_ref[r], 0.0)
        f = f + jnp.dot(h, wf1_ref[r], preferred_element_type=jnp.float32)
    g = jnp.maximum(f + bf1_ref[...], 0.0)
    out_ref[...] = jnp.maximum(
        jnp.dot(g, wf2_ref[...], preferred_element_type=jnp.float32)
        + bf2_ref[...], 0.0)


def _mlp(acc, x, W1, b1, W2, b2, Wf1, bf1, Wf2, bf2):
    BN = 1000
    grid = (N // BN,)
    acc3 = acc.reshape(R, SLAB, D)
    return pl.pallas_call(
        _mlp_kernel,
        grid=grid,
        in_specs=[
            pl.BlockSpec((R, BN, D), lambda i: (0, i, 0)),
            pl.BlockSpec((BN, D), lambda i: (i, 0)),
            pl.BlockSpec((R, D, D), lambda i: (0, 0, 0)),
            pl.BlockSpec((R, 1, D), lambda i: (0, 0, 0)),
            pl.BlockSpec((R, D, D), lambda i: (0, 0, 0)),
            pl.BlockSpec((R, 1, D), lambda i: (0, 0, 0)),
            pl.BlockSpec((R, D, D), lambda i: (0, 0, 0)),
            pl.BlockSpec((1, D), lambda i: (0, 0)),
            pl.BlockSpec((D, D), lambda i: (0, 0)),
            pl.BlockSpec((1, D), lambda i: (0, 0)),
        ],
        out_specs=pl.BlockSpec((BN, D), lambda i: (i, 0)),
        out_shape=jax.ShapeDtypeStruct((N, D), jnp.float32),
    )(acc3, x, W1, b1.reshape(R, 1, D), W2, b2.reshape(R, 1, D),
      Wf1.reshape(R, D, D), bf1.reshape(1, D), Wf2, bf2.reshape(1, D))


def kernel(x, edge_index, edge_type, cell_dimensions,
           W1, b1, W2, b2, Wf1, bf1, Wf2, bf2):
    del cell_dimensions
    src = edge_index[0]
    dst = edge_index[1]
    pad = E_PAD - E
    srcp = jnp.concatenate([src, jnp.zeros((pad,), jnp.int32)])
    comb = edge_type * SLAB + dst
    combp = jnp.concatenate([comb, jnp.full((pad,), DUMMY_ROW, jnp.int32)])
    gidx = (srcp * NCB)[None, :] + jnp.arange(NCB, dtype=jnp.int32)[:, None]
    gidx = gidx.reshape(NCB, NTILES, CHUNKS, K)
    sidx = combp.reshape(NTILES, CHUNKS, K)

    acc = _sc_scatter(x, gidx, sidx)
    return _mlp(acc, x, W1, b1, W2, b2, Wf1, bf1, Wf2, bf2)
